# baseline (speedup 1.0000x reference)
import functools

import jax
import jax.numpy as jnp
from jax.experimental import pallas as pl
from jax.experimental.pallas import tpu as pltpu

_H = 32
_FL = 6
_GL = 6


def _body(xb_ref, w1_ref, fb1_ref, fbh_ref, bd_ref, gw_ref, gb_ref,
          out_ref, *, n):
    ts = out_ref.shape[2]
    bf16 = jnp.bfloat16
    xb = xb_ref[...].reshape(ts, 4, 128)
    x0123 = jnp.concatenate(
        [xb[:, 0, :], xb[:, 1, :], xb[:, 2, :], xb[:, 3, :]],
        axis=1).astype(bf16)

    acc = None
    for gidx in range(4):
        hg = []
        for k in range(4):
            u = 4 * gidx + k
            h = jnp.dot(x0123, w1_ref[:, 256 * u:256 * (u + 1)],
                        preferred_element_type=jnp.float32)
            hg.append(jnp.maximum(
                h.astype(bf16) + fb1_ref[0:1, 256 * u:256 * (u + 1)],
                jnp.zeros((), bf16)))
        for i in range(_FL - 1):
            for k in range(4):
                h32 = jnp.dot(hg[k], bd_ref[256 * i:256 * (i + 1), :],
                              preferred_element_type=jnp.float32)
                hg[k] = jnp.maximum(h32.astype(bf16) + fbh_ref[i:i + 1, :],
                                    jnp.zeros((), bf16))
        i = _FL - 1
        for k in range(4):
            h32 = jnp.dot(hg[k], bd_ref[256 * i:256 * (i + 1), :],
                          preferred_element_type=jnp.float32)
            t = jnp.maximum(h32 + fbh_ref[i:i + 1, :].astype(jnp.float32),
                            0.0)
            acc = t if acc is None else acc + t

    l = acc[:, 0:128] + acc[:, 128:256]
    l = l[:, 0:32] + l[:, 32:64] + l[:, 64:96] + l[:, 96:128]
    p = l * (1.0 / n)

    s = p.T
    for i in range(_GL + 1):
        s = jnp.dot(gw_ref[_H * i:_H * (i + 1), :], s,
                    preferred_element_type=jnp.float32) + gb_ref[:, i:i + 1]
        s = jnp.maximum(s, 0.0)
    fin = jnp.dot(gw_ref[_H * 7:_H * 7 + 8, :], s,
                  preferred_element_type=jnp.float32)
    out_ref[...] = (fin[0:1, :] + gb_ref[0:1, 7:8]).reshape(1, 1, ts)


def kernel(x, f_w1, f_b1, f_res_w, f_res_b, f_wlast, f_blast,
           g_res_w, g_res_b, g_w1, g_b1, g_w2, g_b2):
    B, N, F = x.shape
    assert N == 128 and F == 4, "packing assumes N=128, F=4"
    f32 = jnp.float32

    bf16 = jnp.bfloat16
    eye128 = jnp.eye(128, dtype=bf16)
    w1map = jnp.concatenate(
        [jnp.kron(eye128, f_w1[c:c + 1, :].astype(bf16)) for c in range(F)],
        axis=0)
    fb1 = jnp.tile(f_b1.astype(bf16), (1, 128))
    eye8 = jnp.eye(8, dtype=bf16)
    bd = jnp.concatenate([jnp.kron(eye8, f_res_w[i].astype(bf16))
                          for i in range(_FL)], axis=0)
    fbh = jnp.concatenate(
        [jnp.tile(f_res_b[i:i + 1].astype(bf16), (1, 8)) for i in range(_FL)]
        + [jnp.zeros((2, 256), bf16)], axis=0)

    w_gc = jnp.dot(f_wlast, g_res_w[0])
    b_gc = jnp.dot(f_blast, g_res_w[0]) + g_res_b[0:1]
    gw = jnp.concatenate(
        [w_gc.T] + [g_res_w[i].astype(f32).T for i in range(1, _GL)]
        + [g_w1.astype(f32).T, g_w2.astype(f32).T, jnp.zeros((7, _H), f32)],
        axis=0)
    gb = jnp.concatenate(
        [b_gc, g_res_b[1:_GL].astype(f32), g_b1.astype(f32),
         jnp.pad(g_b2.astype(f32), ((0, 0), (0, _H - 1)))], axis=0).T

    xt = x.transpose(0, 2, 1).reshape(B * F, N)

    TS = min(512, B)
    out = pl.pallas_call(
        functools.partial(_body, n=N),
        out_shape=jax.ShapeDtypeStruct((B // TS, 1, TS), f32),
        grid=(B // TS,),
        in_specs=[
            pl.BlockSpec((F * TS, N), lambda i: (i, 0)),
            pl.BlockSpec((F * N, 4096), lambda i: (0, 0)),
            pl.BlockSpec((1, 4096), lambda i: (0, 0)),
            pl.BlockSpec((8, 256), lambda i: (0, 0)),
            pl.BlockSpec((1536, 256), lambda i: (0, 0)),
            pl.BlockSpec((232, _H), lambda i: (0, 0)),
            pl.BlockSpec((_H, 8), lambda i: (0, 0)),
        ],
        out_specs=pl.BlockSpec((1, 1, TS), lambda i: (i, 0, 0)),
        compiler_params=pltpu.CompilerParams(
            dimension_semantics=("parallel",)),
    )(xt, w1map, fb1, fbh, bd, gw, gb)
    return out.reshape(B)


# speedup vs baseline: 1.0428x; 1.0428x over previous
import functools

import jax
import jax.numpy as jnp
from jax.experimental import pallas as pl
from jax.experimental.pallas import tpu as pltpu

_H = 32
_FL = 6
_GL = 6


def _body(xb_ref, w1_ref, fb1_ref, fbh_ref, bd_ref, gw_ref, gb_ref,
          out_ref, *, n):
    ts = out_ref.shape[2]
    bf16 = jnp.bfloat16
    xb = xb_ref[...].reshape(ts, 4, 128)
    x0123 = jnp.concatenate(
        [xb[:, 0, :], xb[:, 1, :], xb[:, 2, :], xb[:, 3, :]],
        axis=1).astype(bf16)

    acc = None
    for gidx in range(4):
        hg = []
        for k in range(4):
            u = 4 * gidx + k
            h = jnp.dot(x0123, w1_ref[:, 256 * u:256 * (u + 1)],
                        preferred_element_type=jnp.float32)
            hg.append(jnp.maximum(
                h.astype(bf16) + fb1_ref[0:1, 256 * u:256 * (u + 1)],
                jnp.zeros((), bf16)))
        for i in range(_FL - 1):
            for k in range(4):
                h32 = jnp.dot(hg[k], bd_ref[256 * i:256 * (i + 1), :],
                              preferred_element_type=jnp.float32)
                hg[k] = jnp.maximum(h32.astype(bf16) + fbh_ref[i:i + 1, :],
                                    jnp.zeros((), bf16))
        i = _FL - 1
        for k in range(4):
            h32 = jnp.dot(hg[k], bd_ref[256 * i:256 * (i + 1), :],
                          preferred_element_type=jnp.float32)
            t = jnp.maximum(h32 + fbh_ref[i:i + 1, :].astype(jnp.float32),
                            0.0)
            acc = t if acc is None else acc + t

    l = acc[:, 0:128] + acc[:, 128:256]
    l = l[:, 0:32] + l[:, 32:64] + l[:, 64:96] + l[:, 96:128]
    p = l * (1.0 / n)

    s = p.T
    for i in range(_GL + 1):
        s = jnp.dot(gw_ref[_H * i:_H * (i + 1), :], s,
                    preferred_element_type=jnp.float32) + gb_ref[:, i:i + 1]
        s = jnp.maximum(s, 0.0)
    fin = jnp.dot(gw_ref[_H * 7:_H * 7 + 8, :], s,
                  preferred_element_type=jnp.float32)
    out_ref[...] = (fin[0:1, :] + gb_ref[0:1, 7:8]).reshape(1, 1, ts)


def kernel(x, f_w1, f_b1, f_res_w, f_res_b, f_wlast, f_blast,
           g_res_w, g_res_b, g_w1, g_b1, g_w2, g_b2):
    B, N, F = x.shape
    assert N == 128 and F == 4, "packing assumes N=128, F=4"
    f32 = jnp.float32

    bf16 = jnp.bfloat16
    eye128 = jnp.eye(128, dtype=bf16)
    w1map = (f_w1.astype(bf16)[:, None, None, :]
             * eye128[None, :, :, None]).reshape(512, 4096)
    fb1 = jnp.tile(f_b1.astype(bf16), (1, 128))
    eye8 = jnp.eye(8, dtype=bf16)
    bd = (f_res_w.astype(bf16)[:, None, :, None, :]
          * eye8[None, :, None, :, None]).reshape(1536, 256)
    fbh = jnp.concatenate(
        [jnp.broadcast_to(f_res_b.astype(bf16)[:, None, :],
                          (_FL, 8, _H)).reshape(_FL, 256),
         jnp.zeros((2, 256), bf16)], axis=0)

    w_gc = jnp.dot(f_wlast, g_res_w[0])
    b_gc = jnp.dot(f_blast, g_res_w[0]) + g_res_b[0:1]
    gw = jnp.concatenate(
        [w_gc.T] + [g_res_w[i].astype(f32).T for i in range(1, _GL)]
        + [g_w1.astype(f32).T, g_w2.astype(f32).T, jnp.zeros((7, _H), f32)],
        axis=0)
    gb = jnp.concatenate(
        [b_gc, g_res_b[1:_GL].astype(f32), g_b1.astype(f32),
         jnp.pad(g_b2.astype(f32), ((0, 0), (0, _H - 1)))], axis=0).T

    xt = x.transpose(0, 2, 1).reshape(B * F, N)

    TS = min(1024, B)
    out = pl.pallas_call(
        functools.partial(_body, n=N),
        out_shape=jax.ShapeDtypeStruct((B // TS, 1, TS), f32),
        grid=(B // TS,),
        in_specs=[
            pl.BlockSpec((F * TS, N), lambda i: (i, 0)),
            pl.BlockSpec((F * N, 4096), lambda i: (0, 0)),
            pl.BlockSpec((1, 4096), lambda i: (0, 0)),
            pl.BlockSpec((8, 256), lambda i: (0, 0)),
            pl.BlockSpec((1536, 256), lambda i: (0, 0)),
            pl.BlockSpec((232, _H), lambda i: (0, 0)),
            pl.BlockSpec((_H, 8), lambda i: (0, 0)),
        ],
        out_specs=pl.BlockSpec((1, 1, TS), lambda i: (i, 0, 0)),
        compiler_params=pltpu.CompilerParams(
            dimension_semantics=("parallel",)),
    )(xt, w1map, fb1, fbh, bd, gw, gb)
    return out.reshape(B)


# speedup vs baseline: 1.0656x; 1.0218x over previous
import functools

import jax
import jax.numpy as jnp
from jax.experimental import pallas as pl
from jax.experimental.pallas import tpu as pltpu

_H = 32
_FL = 6
_GL = 6


def _body(xb_ref, w1_ref, fb1_ref, fbh_ref, bd_ref, gw_ref, gb_ref,
          out_ref, *, n):
    ts = out_ref.shape[2]
    bf16 = jnp.bfloat16
    xb = xb_ref[...].reshape(ts, 4, 128)
    x0123 = jnp.concatenate(
        [xb[:, 0, :], xb[:, 1, :], xb[:, 2, :], xb[:, 3, :]],
        axis=1).astype(bf16)

    acc = None
    for gidx in range(2):
        hg = []
        for k in range(8):
            u = 8 * gidx + k
            h = jnp.dot(x0123, w1_ref[:, 256 * u:256 * (u + 1)],
                        preferred_element_type=jnp.float32)
            hg.append(jnp.maximum(
                h.astype(bf16) + fb1_ref[0:1, 256 * u:256 * (u + 1)],
                jnp.zeros((), bf16)))
        for i in range(_FL - 1):
            for k in range(8):
                h32 = jnp.dot(hg[k], bd_ref[256 * i:256 * (i + 1), :],
                              preferred_element_type=jnp.float32)
                hg[k] = jnp.maximum(h32.astype(bf16) + fbh_ref[i:i + 1, :],
                                    jnp.zeros((), bf16))
        i = _FL - 1
        for k in range(8):
            h32 = jnp.dot(hg[k], bd_ref[256 * i:256 * (i + 1), :],
                          preferred_element_type=jnp.float32)
            t = jnp.maximum(h32 + fbh_ref[i:i + 1, :].astype(jnp.float32),
                            0.0)
            acc = t if acc is None else acc + t

    l = acc[:, 0:128] + acc[:, 128:256]
    l = l[:, 0:32] + l[:, 32:64] + l[:, 64:96] + l[:, 96:128]
    p = l * (1.0 / n)

    s = p.T
    for i in range(_GL + 1):
        s = jnp.dot(gw_ref[_H * i:_H * (i + 1), :], s,
                    preferred_element_type=jnp.float32) + gb_ref[:, i:i + 1]
        s = jnp.maximum(s, 0.0)
    fin = jnp.dot(gw_ref[_H * 7:_H * 7 + 8, :], s,
                  preferred_element_type=jnp.float32)
    out_ref[...] = (fin[0:1, :] + gb_ref[0:1, 7:8]).reshape(1, 1, ts)


def kernel(x, f_w1, f_b1, f_res_w, f_res_b, f_wlast, f_blast,
           g_res_w, g_res_b, g_w1, g_b1, g_w2, g_b2):
    B, N, F = x.shape
    assert N == 128 and F == 4, "packing assumes N=128, F=4"
    f32 = jnp.float32

    bf16 = jnp.bfloat16
    eye128 = jnp.eye(128, dtype=bf16)
    w1map = (f_w1.astype(bf16)[:, None, None, :]
             * eye128[None, :, :, None]).reshape(512, 4096)
    fb1 = jnp.tile(f_b1.astype(bf16), (1, 128))
    eye8 = jnp.eye(8, dtype=bf16)
    bd = (f_res_w.astype(bf16)[:, None, :, None, :]
          * eye8[None, :, None, :, None]).reshape(1536, 256)
    fbh = jnp.concatenate(
        [jnp.broadcast_to(f_res_b.astype(bf16)[:, None, :],
                          (_FL, 8, _H)).reshape(_FL, 256),
         jnp.zeros((2, 256), bf16)], axis=0)

    w_gc = jnp.dot(f_wlast, g_res_w[0])
    b_gc = jnp.dot(f_blast, g_res_w[0]) + g_res_b[0:1]
    gw = jnp.concatenate(
        [w_gc.T] + [g_res_w[i].astype(f32).T for i in range(1, _GL)]
        + [g_w1.astype(f32).T, g_w2.astype(f32).T, jnp.zeros((7, _H), f32)],
        axis=0)
    gb = jnp.concatenate(
        [b_gc, g_res_b[1:_GL].astype(f32), g_b1.astype(f32),
         jnp.pad(g_b2.astype(f32), ((0, 0), (0, _H - 1)))], axis=0).T

    xt = x.transpose(0, 2, 1).reshape(B * F, N)

    TS = min(1024, B)
    out = pl.pallas_call(
        functools.partial(_body, n=N),
        out_shape=jax.ShapeDtypeStruct((B // TS, 1, TS), f32),
        grid=(B // TS,),
        in_specs=[
            pl.BlockSpec((F * TS, N), lambda i: (i, 0)),
            pl.BlockSpec((F * N, 4096), lambda i: (0, 0)),
            pl.BlockSpec((1, 4096), lambda i: (0, 0)),
            pl.BlockSpec((8, 256), lambda i: (0, 0)),
            pl.BlockSpec((1536, 256), lambda i: (0, 0)),
            pl.BlockSpec((232, _H), lambda i: (0, 0)),
            pl.BlockSpec((_H, 8), lambda i: (0, 0)),
        ],
        out_specs=pl.BlockSpec((1, 1, TS), lambda i: (i, 0, 0)),
        compiler_params=pltpu.CompilerParams(
            dimension_semantics=("parallel",)),
    )(xt, w1map, fb1, fbh, bd, gw, gb)
    return out.reshape(B)
